# baseline (device time: 18921 ns/iter reference)
import jax
import jax.numpy as jnp
from jax import lax
from jax.experimental import pallas as pl
from jax.experimental.pallas import tpu as pltpu

N_CHUNKS = 4


def kernel(ids, E):
    T = ids.shape[0]
    V_local, D = E.shape
    C = T // N_CHUNKS

    ids2 = ids.reshape(T, 1)

    def body(ids_ref, e_ref, out_ref, pbuf, ybuf, gbuf, send_sems, recv_sems):
        my_x = lax.axis_index("x")
        my_y = lax.axis_index("y")
        my_z = lax.axis_index("z")
        y_partner = (my_x, 1 - my_y, my_z)
        x_partner = (1 - my_x, my_y, my_z)
        z_partner = (my_x, my_y, 1 - my_z)

        barrier = pltpu.get_barrier_semaphore()
        for p in (y_partner, x_partner, z_partner):
            pl.semaphore_signal(
                barrier, inc=1, device_id=p,
                device_id_type=pl.DeviceIdType.MESH,
            )
        pl.semaphore_wait(barrier, 3)

        c0 = 2 * my_x + my_z
        c1 = 2 * (1 - my_x) + my_z
        c2 = 2 * my_x + (1 - my_z)
        c3 = 2 * (1 - my_x) + (1 - my_z)

        my_ids = ids_ref[pl.ds(c0 * C, C), :] - my_y * V_local
        iota = lax.broadcasted_iota(jnp.int32, (C, V_local), 1)
        onehot = (iota == my_ids).astype(jnp.bfloat16)
        partial = jnp.dot(
            onehot, e_ref[:, :].astype(jnp.bfloat16),
            preferred_element_type=jnp.float32,
        )
        pbuf[:, :] = partial.astype(jnp.bfloat16)

        y_rdma = pltpu.make_async_remote_copy(
            src_ref=pbuf, dst_ref=ybuf,
            send_sem=send_sems.at[0], recv_sem=recv_sems.at[0],
            device_id=y_partner, device_id_type=pl.DeviceIdType.MESH,
        )
        y_rdma.start()
        y_rdma.wait()
        gbuf[0, :, :] = pbuf[:, :] + ybuf[:, :]

        x_rdma = pltpu.make_async_remote_copy(
            src_ref=gbuf.at[0], dst_ref=gbuf.at[1],
            send_sem=send_sems.at[1], recv_sem=recv_sems.at[1],
            device_id=x_partner, device_id_type=pl.DeviceIdType.MESH,
        )
        za_rdma = pltpu.make_async_remote_copy(
            src_ref=gbuf.at[0], dst_ref=gbuf.at[2],
            send_sem=send_sems.at[2], recv_sem=recv_sems.at[2],
            device_id=z_partner, device_id_type=pl.DeviceIdType.MESH,
        )
        x_rdma.start()
        za_rdma.start()

        x_rdma.wait()
        zb_rdma = pltpu.make_async_remote_copy(
            src_ref=gbuf.at[1], dst_ref=gbuf.at[3],
            send_sem=send_sems.at[3], recv_sem=recv_sems.at[3],
            device_id=z_partner, device_id_type=pl.DeviceIdType.MESH,
        )
        zb_rdma.start()

        out_ref[pl.ds(c0 * C, C), :] = gbuf[0, :, :].astype(jnp.float32)
        out_ref[pl.ds(c1 * C, C), :] = gbuf[1, :, :].astype(jnp.float32)

        za_rdma.wait()
        out_ref[pl.ds(c2 * C, C), :] = gbuf[2, :, :].astype(jnp.float32)
        zb_rdma.wait()
        out_ref[pl.ds(c3 * C, C), :] = gbuf[3, :, :].astype(jnp.float32)

    return pl.pallas_call(
        body,
        out_shape=jax.ShapeDtypeStruct((T, D), jnp.float32),
        in_specs=[
            pl.BlockSpec(memory_space=pltpu.VMEM),
            pl.BlockSpec(memory_space=pltpu.VMEM),
        ],
        out_specs=pl.BlockSpec(memory_space=pltpu.VMEM),
        scratch_shapes=[
            pltpu.VMEM((C, D), jnp.bfloat16),
            pltpu.VMEM((C, D), jnp.bfloat16),
            pltpu.VMEM((N_CHUNKS, C, D), jnp.bfloat16),
            pltpu.SemaphoreType.DMA((4,)),
            pltpu.SemaphoreType.DMA((4,)),
        ],
        compiler_params=pltpu.CompilerParams(collective_id=0),
    )(ids2, E)


# device time: 9197 ns/iter; 2.0573x vs baseline; 2.0573x over previous
import os

import jax
import jax.numpy as jnp
from jax import lax
from jax.experimental import pallas as pl
from jax.experimental.pallas import tpu as pltpu

N_CHUNKS = 4
_COMPUTE_ONLY = os.environ.get("KERNEL_COMPUTE_ONLY", "0") == "1"


def kernel(ids, E):
    T = ids.shape[0]
    V_local, D = E.shape
    C = T // N_CHUNKS

    ids2 = ids.reshape(T, 1)

    def body(ids_ref, e_ref, out_ref, pbuf, ybuf, gbuf, send_sems, recv_sems):
        my_x = lax.axis_index("x")
        my_y = lax.axis_index("y")
        my_z = lax.axis_index("z")
        y_partner = (my_x, 1 - my_y, my_z)
        x_partner = (1 - my_x, my_y, my_z)
        z_partner = (my_x, my_y, 1 - my_z)

        barrier = pltpu.get_barrier_semaphore()
        for p in (y_partner, x_partner, z_partner):
            pl.semaphore_signal(
                barrier, inc=1, device_id=p,
                device_id_type=pl.DeviceIdType.MESH,
            )
        pl.semaphore_wait(barrier, 3)

        c0 = 2 * my_x + my_z
        c1 = 2 * (1 - my_x) + my_z
        c2 = 2 * my_x + (1 - my_z)
        c3 = 2 * (1 - my_x) + (1 - my_z)

        my_ids = ids_ref[pl.ds(c0 * C, C), :] - my_y * V_local
        iota = lax.broadcasted_iota(jnp.int32, (C, V_local), 1)
        onehot = (iota == my_ids).astype(jnp.bfloat16)
        partial = jnp.dot(
            onehot, e_ref[:, :].astype(jnp.bfloat16),
            preferred_element_type=jnp.float32,
        )
        pbuf[:, :] = partial.astype(jnp.bfloat16)

        if _COMPUTE_ONLY:
            out_ref[:, :] = jnp.zeros((T, D), jnp.float32)
            out_ref[pl.ds(c0 * C, C), :] = pbuf[:, :].astype(jnp.float32)
            return

        y_rdma = pltpu.make_async_remote_copy(
            src_ref=pbuf, dst_ref=ybuf,
            send_sem=send_sems.at[0], recv_sem=recv_sems.at[0],
            device_id=y_partner, device_id_type=pl.DeviceIdType.MESH,
        )
        y_rdma.start()
        y_rdma.wait()
        gbuf[0, :, :] = pbuf[:, :] + ybuf[:, :]

        x_rdma = pltpu.make_async_remote_copy(
            src_ref=gbuf.at[0], dst_ref=gbuf.at[1],
            send_sem=send_sems.at[1], recv_sem=recv_sems.at[1],
            device_id=x_partner, device_id_type=pl.DeviceIdType.MESH,
        )
        za_rdma = pltpu.make_async_remote_copy(
            src_ref=gbuf.at[0], dst_ref=gbuf.at[2],
            send_sem=send_sems.at[2], recv_sem=recv_sems.at[2],
            device_id=z_partner, device_id_type=pl.DeviceIdType.MESH,
        )
        x_rdma.start()
        za_rdma.start()

        x_rdma.wait()
        zb_rdma = pltpu.make_async_remote_copy(
            src_ref=gbuf.at[1], dst_ref=gbuf.at[3],
            send_sem=send_sems.at[3], recv_sem=recv_sems.at[3],
            device_id=z_partner, device_id_type=pl.DeviceIdType.MESH,
        )
        zb_rdma.start()

        out_ref[pl.ds(c0 * C, C), :] = gbuf[0, :, :].astype(jnp.float32)
        out_ref[pl.ds(c1 * C, C), :] = gbuf[1, :, :].astype(jnp.float32)

        za_rdma.wait()
        out_ref[pl.ds(c2 * C, C), :] = gbuf[2, :, :].astype(jnp.float32)
        zb_rdma.wait()
        out_ref[pl.ds(c3 * C, C), :] = gbuf[3, :, :].astype(jnp.float32)

    return pl.pallas_call(
        body,
        out_shape=jax.ShapeDtypeStruct((T, D), jnp.float32),
        in_specs=[
            pl.BlockSpec(memory_space=pltpu.VMEM),
            pl.BlockSpec(memory_space=pltpu.VMEM),
        ],
        out_specs=pl.BlockSpec(memory_space=pltpu.VMEM),
        scratch_shapes=[
            pltpu.VMEM((C, D), jnp.bfloat16),
            pltpu.VMEM((C, D), jnp.bfloat16),
            pltpu.VMEM((N_CHUNKS, C, D), jnp.bfloat16),
            pltpu.SemaphoreType.DMA((4,)),
            pltpu.SemaphoreType.DMA((4,)),
        ],
        compiler_params=pltpu.CompilerParams(collective_id=0),
    )(ids2, E)


# device time: 9122 ns/iter; 2.0742x vs baseline; 1.0082x over previous
import os

import jax
import jax.numpy as jnp
from jax import lax
from jax.experimental import pallas as pl
from jax.experimental.pallas import tpu as pltpu

N_CHUNKS = 4
_COMPUTE_ONLY = os.environ.get("KERNEL_COMPUTE_ONLY", "0") == "1"
_COMPUTE_MODE = os.environ.get("KERNEL_COMPUTE_MODE", "bf16")


def kernel(ids, E):
    T = ids.shape[0]
    V_local, D = E.shape
    C = T // N_CHUNKS

    ids2 = ids.reshape(T, 1)

    def body(ids_ref, e_ref, out_ref, pbuf, ybuf, gbuf, send_sems, recv_sems):
        my_x = lax.axis_index("x")
        my_y = lax.axis_index("y")
        my_z = lax.axis_index("z")
        y_partner = (my_x, 1 - my_y, my_z)
        x_partner = (1 - my_x, my_y, my_z)
        z_partner = (my_x, my_y, 1 - my_z)

        barrier = pltpu.get_barrier_semaphore()
        for p in (y_partner, x_partner, z_partner):
            pl.semaphore_signal(
                barrier, inc=1, device_id=p,
                device_id_type=pl.DeviceIdType.MESH,
            )
        pl.semaphore_wait(barrier, 3)

        c0 = 2 * my_x + my_z
        c1 = 2 * (1 - my_x) + my_z
        c2 = 2 * my_x + (1 - my_z)
        c3 = 2 * (1 - my_x) + (1 - my_z)

        if _COMPUTE_MODE == "loop":
            def gather_row(i, carry):
                idx = ids_ref[c0 * C + i, 0] - my_y * V_local
                valid = jnp.logical_and(idx >= 0, idx < V_local)
                safe = jnp.clip(idx, 0, V_local - 1)
                row = e_ref[pl.ds(safe, 1), :]
                pbuf[pl.ds(i, 1), :] = jnp.where(valid, row, 0.0).astype(
                    jnp.bfloat16
                )
                return carry

            lax.fori_loop(0, C, gather_row, 0)
        else:
            my_ids = ids_ref[pl.ds(c0 * C, C), :] - my_y * V_local
            iota = lax.broadcasted_iota(jnp.int32, (C, V_local), 1)
            if _COMPUTE_MODE == "f32":
                onehot = (iota == my_ids).astype(jnp.float32)
                partial = jnp.dot(
                    onehot, e_ref[:, :], preferred_element_type=jnp.float32
                )
            else:
                onehot = (iota == my_ids).astype(jnp.bfloat16)
                partial = jnp.dot(
                    onehot, e_ref[:, :].astype(jnp.bfloat16),
                    preferred_element_type=jnp.float32,
                )
            pbuf[:, :] = partial.astype(jnp.bfloat16)

        if _COMPUTE_ONLY:
            out_ref[:, :] = jnp.zeros((T, D), jnp.float32)
            out_ref[pl.ds(c0 * C, C), :] = pbuf[:, :].astype(jnp.float32)
            return

        y_rdma = pltpu.make_async_remote_copy(
            src_ref=pbuf, dst_ref=ybuf,
            send_sem=send_sems.at[0], recv_sem=recv_sems.at[0],
            device_id=y_partner, device_id_type=pl.DeviceIdType.MESH,
        )
        y_rdma.start()
        y_rdma.wait()
        gbuf[0, :, :] = pbuf[:, :] + ybuf[:, :]

        x_rdma = pltpu.make_async_remote_copy(
            src_ref=gbuf.at[0], dst_ref=gbuf.at[1],
            send_sem=send_sems.at[1], recv_sem=recv_sems.at[1],
            device_id=x_partner, device_id_type=pl.DeviceIdType.MESH,
        )
        za_rdma = pltpu.make_async_remote_copy(
            src_ref=gbuf.at[0], dst_ref=gbuf.at[2],
            send_sem=send_sems.at[2], recv_sem=recv_sems.at[2],
            device_id=z_partner, device_id_type=pl.DeviceIdType.MESH,
        )
        x_rdma.start()
        za_rdma.start()

        x_rdma.wait()
        zb_rdma = pltpu.make_async_remote_copy(
            src_ref=gbuf.at[1], dst_ref=gbuf.at[3],
            send_sem=send_sems.at[3], recv_sem=recv_sems.at[3],
            device_id=z_partner, device_id_type=pl.DeviceIdType.MESH,
        )
        zb_rdma.start()

        out_ref[pl.ds(c0 * C, C), :] = gbuf[0, :, :].astype(jnp.float32)
        out_ref[pl.ds(c1 * C, C), :] = gbuf[1, :, :].astype(jnp.float32)

        za_rdma.wait()
        out_ref[pl.ds(c2 * C, C), :] = gbuf[2, :, :].astype(jnp.float32)
        zb_rdma.wait()
        out_ref[pl.ds(c3 * C, C), :] = gbuf[3, :, :].astype(jnp.float32)

    return pl.pallas_call(
        body,
        out_shape=jax.ShapeDtypeStruct((T, D), jnp.float32),
        in_specs=[
            pl.BlockSpec(
                memory_space=pltpu.SMEM
                if _COMPUTE_MODE == "loop"
                else pltpu.VMEM
            ),
            pl.BlockSpec(memory_space=pltpu.VMEM),
        ],
        out_specs=pl.BlockSpec(memory_space=pltpu.VMEM),
        scratch_shapes=[
            pltpu.VMEM((C, D), jnp.bfloat16),
            pltpu.VMEM((C, D), jnp.bfloat16),
            pltpu.VMEM((N_CHUNKS, C, D), jnp.bfloat16),
            pltpu.SemaphoreType.DMA((4,)),
            pltpu.SemaphoreType.DMA((4,)),
        ],
        compiler_params=pltpu.CompilerParams(collective_id=0),
    )(ids2, E)
